# baseline (speedup 1.0000x reference)
import functools

import jax
import jax.numpy as jnp
from jax import lax
from jax.experimental import pallas as pl
from jax.experimental.pallas import tpu as pltpu
from jax.experimental.pallas import tpu_sc as plsc

N = 100000
NP = 102400
E = 3200000
EP = 3276800
EPM = EP // 128
G = 64
NC, NS = 2, 16
NTILES = NC * NS
K = 8
NSL = NP // NS
DUMMY = NP - 1
EDGE_SPLIT = (984, 616)

F32 = jnp.float32


def _sc_mesh():
    return plsc.VectorSubcoreMesh(core_axis_name="c", subcore_axis_name="s")


_SC_PARAMS = pltpu.CompilerParams(use_tc_tiling_on_sc=False)


def _sc_degree(dstp, zeros1, ones_rows):
    @functools.partial(
        pl.kernel,
        out_type=[jax.ShapeDtypeStruct((NP,), F32) for _ in range(NC)],
        mesh=_sc_mesh(),
        compiler_params=_SC_PARAMS,
        scratch_types=[
            pltpu.VMEM_SHARED((NP,), F32),
            pltpu.VMEM((K, 128), jnp.int32),
            pltpu.VMEM((K, 128), F32),
            pltpu.SemaphoreType.DMA,
        ],
    )
    def k(dstp_h, z_h, ones_h, dega_h, degb_h, acc, idxd, ones_v, ssem):
        c = lax.axis_index("c")
        s = lax.axis_index("s")
        sl = pl.ds(s * NSL, NSL)
        pltpu.sync_copy(z_h.at[sl], acc.at[sl])
        pltpu.sync_copy(ones_h, ones_v)
        plsc.subcore_barrier()
        tb = (c * NS + s) * (EPM // NTILES)

        def body(kk, carry):
            base = tb + kk * K
            pltpu.sync_copy(dstp_h.at[pl.ds(base, K)], idxd)
            cps = [
                pltpu.async_copy(ones_v.at[j], acc.at[idxd.at[j]], ssem, add=True)
                for j in range(K)
            ]
            for cp in cps:
                cp.wait()
            return carry

        lax.fori_loop(0, EPM // NTILES // K, body, 0)
        plsc.subcore_barrier()

        @pl.when(c == 0)
        def _():
            pltpu.sync_copy(acc.at[sl], dega_h.at[sl])

        @pl.when(c == 1)
        def _():
            pltpu.sync_copy(acc.at[sl], degb_h.at[sl])

    return k(dstp, zeros1, ones_rows)


def _sc_edge_pass(table, srcp, dstp, zeros, R):
    KC = 4
    R0, R1 = EDGE_SPLIT
    assert 16 * (R0 + R1) == EPM and R0 % 8 == 0 and R1 % 8 == 0

    @functools.partial(
        pl.kernel,
        out_type=[jax.ShapeDtypeStruct((NP, R), F32) for _ in range(NC)],
        mesh=_sc_mesh(),
        compiler_params=_SC_PARAMS,
        scratch_types=[
            pltpu.VMEM_SHARED((NP, R), F32),
            [pltpu.VMEM((KC, 128), jnp.int32) for _ in range(2)],
            [pltpu.VMEM((KC, 128), jnp.int32) for _ in range(2)],
            [pltpu.VMEM((KC, 128, R), F32) for _ in range(2)],
            pltpu.SemaphoreType.DMA,
            pltpu.SemaphoreType.DMA,
            pltpu.SemaphoreType.DMA,
        ],
    )
    def k(table_h, srcp_h, dstp_h, z_h, outa_h, outb_h,
          acc, idxs, idxd, rows, gsem, ssem, isem):
        c = lax.axis_index("c")
        s = lax.axis_index("s")
        sl = pl.ds(s * NSL, NSL)
        pltpu.sync_copy(z_h.at[sl], acc.at[sl])
        plsc.subcore_barrier()
        tb = jnp.where(c == 0, s * R0, NS * R0 + s * R1)
        nch = jnp.where(c == 0, R0 // KC, R1 // KC)

        def fire_idx(kk, b):
            base = tb + kk * KC
            pltpu.async_copy(srcp_h.at[pl.ds(base, KC)], idxs[b], isem)
            pltpu.async_copy(dstp_h.at[pl.ds(base, KC)], idxd[b], isem)

        def wait_idx(b):
            pltpu.make_async_copy(srcp_h.at[pl.ds(0, KC)], idxs[b], isem).wait()
            pltpu.make_async_copy(dstp_h.at[pl.ds(0, KC)], idxd[b], isem).wait()

        def fire_gather(b):
            for j in range(KC):
                pltpu.async_copy(table_h.at[idxs[b].at[j]], rows[b].at[j], gsem)

        def wait_gather(b):
            for j in range(KC):
                pltpu.make_async_copy(
                    table_h.at[idxs[b].at[j]], rows[b].at[j], gsem).wait()

        def fire_scatter(b):
            for j in range(KC):
                pltpu.async_copy(rows[b].at[j], acc.at[idxd[b].at[j]], ssem,
                                 add=True)

        def wait_scatter(b):
            for j in range(KC):
                pltpu.make_async_copy(
                    rows[b].at[j], acc.at[idxd[b].at[j]], ssem).wait()

        pltpu.sync_copy(srcp_h.at[pl.ds(tb, KC)], idxs[0])
        pltpu.sync_copy(dstp_h.at[pl.ds(tb, KC)], idxd[0])
        fire_gather(0)

        def phase(kk, b):
            @pl.when(kk >= 1)
            def _():
                wait_scatter(b ^ 1)

            @pl.when(kk + 1 < nch)
            def _():
                fire_idx(kk + 1, b ^ 1)

            wait_gather(b)
            fire_scatter(b)

            @pl.when(kk + 1 < nch)
            def _():
                wait_idx(b ^ 1)
                fire_gather(b ^ 1)

        def body(i, carry):
            phase(2 * i, 0)
            phase(2 * i + 1, 1)
            return carry

        lax.fori_loop(0, nch // 2, body, 0)
        wait_scatter(1)
        plsc.subcore_barrier()

        @pl.when(c == 0)
        def _():
            pltpu.sync_copy(acc.at[sl], outa_h.at[sl])

        @pl.when(c == 1)
        def _():
            pltpu.sync_copy(acc.at[sl], outb_h.at[sl])

    return k(table, srcp, dstp, zeros)


def _tc_prolog(dega, degb, x_pad, W1):
    BB = 2048

    def body(da, db, xr, w1, dis_r, xws_r):
        deg = da[...] + db[...] + 1.0
        dis = lax.rsqrt(deg)
        dis_r[...] = dis
        xw = xr[:, 0:1] * w1[0:1, :] + xr[:, 1:2] * w1[1:2, :]
        xws_r[...] = xw * dis

    return pl.pallas_call(
        body,
        grid=(NP // BB,),
        in_specs=[
            pl.BlockSpec((BB, 1), lambda i: (i, 0)),
            pl.BlockSpec((BB, 1), lambda i: (i, 0)),
            pl.BlockSpec((BB, 2), lambda i: (i, 0)),
            pl.BlockSpec((2, 16), lambda i: (0, 0)),
        ],
        out_specs=[
            pl.BlockSpec((BB, 1), lambda i: (i, 0)),
            pl.BlockSpec((BB, 16), lambda i: (i, 0)),
        ],
        out_shape=[
            jax.ShapeDtypeStruct((NP, 1), F32),
            jax.ShapeDtypeStruct((NP, 16), F32),
        ],
    )(dega.reshape(NP, 1), degb.reshape(NP, 1), x_pad, W1)


def _tc_layer1(acc1a, acc1b, xws1, dis, b1):
    BB = 2048

    def body(aa, ab, xws_r, dis_r, b1r, out_r):
        t = aa[...] + ab[...] + xws_r[...]
        h1 = jnp.maximum(t * dis_r[...] + b1r[...], 0.0)
        out_r[...] = h1 * dis_r[...]

    return pl.pallas_call(
        body,
        grid=(NP // BB,),
        in_specs=[
            pl.BlockSpec((BB, 16), lambda i: (i, 0)),
            pl.BlockSpec((BB, 16), lambda i: (i, 0)),
            pl.BlockSpec((BB, 16), lambda i: (i, 0)),
            pl.BlockSpec((BB, 1), lambda i: (i, 0)),
            pl.BlockSpec((1, 16), lambda i: (0, 0)),
        ],
        out_specs=pl.BlockSpec((BB, 16), lambda i: (i, 0)),
        out_shape=jax.ShapeDtypeStruct((NP, 16), F32),
    )(acc1a, acc1b, xws1, dis, b1.reshape(1, 16))


def _tc_final(acc2a, acc2b, h1s, dis, W2, b2, batch3, fc1_W, fc1_b, fc2_W, fc2_b):
    BF = 512
    NBF = NP // BF

    def body(aa, ab, h1s_r, dis_r, w2, b2r, bat, f1w, f1b, f2w, f2b,
             out_r, sums, counts):
        i = pl.program_id(0)

        @pl.when(i == 0)
        def _():
            sums[...] = jnp.zeros_like(sums)
            counts[...] = jnp.zeros_like(counts)

        m = aa[...] + ab[...] + h1s_r[...]
        h2 = jnp.maximum(
            jnp.dot(m, w2[...], preferred_element_type=F32) * dis_r[...]
            + b2r[...], 0.0)
        bid = bat[0, 0, :]
        oh = (bid[None, :] == lax.broadcasted_iota(jnp.int32, (G, BF), 0)
              ).astype(F32)
        sums[...] += jnp.dot(oh, h2, preferred_element_type=F32)
        counts[...] += jnp.sum(oh, axis=1, keepdims=True)

        @pl.when(i == NBF - 1)
        def _():
            pooled = sums[...] / jnp.maximum(counts[...], 1.0)
            hh = jnp.maximum(
                jnp.dot(pooled, f1w[...], preferred_element_type=F32)
                + f1b[...], 0.0)
            out_r[...] = jnp.dot(hh, f2w[...], preferred_element_type=F32) + f2b[...]

    return pl.pallas_call(
        body,
        grid=(NBF,),
        in_specs=[
            pl.BlockSpec((BF, 16), lambda i: (i, 0)),
            pl.BlockSpec((BF, 16), lambda i: (i, 0)),
            pl.BlockSpec((BF, 16), lambda i: (i, 0)),
            pl.BlockSpec((BF, 1), lambda i: (i, 0)),
            pl.BlockSpec((16, 32), lambda i: (0, 0)),
            pl.BlockSpec((1, 32), lambda i: (0, 0)),
            pl.BlockSpec((1, 1, BF), lambda i: (i, 0, 0)),
            pl.BlockSpec((32, 16), lambda i: (0, 0)),
            pl.BlockSpec((1, 16), lambda i: (0, 0)),
            pl.BlockSpec((16, 1), lambda i: (0, 0)),
            pl.BlockSpec((1, 1), lambda i: (0, 0)),
        ],
        out_specs=pl.BlockSpec((G, 1), lambda i: (0, 0)),
        out_shape=jax.ShapeDtypeStruct((G, 1), F32),
        scratch_shapes=[
            pltpu.VMEM((G, 32), F32),
            pltpu.VMEM((G, 1), F32),
        ],
    )(acc2a, acc2b, h1s, dis, W2, b2.reshape(1, 32), batch3,
      fc1_W, fc1_b.reshape(1, 16), fc2_W, fc2_b.reshape(1, 1))


def kernel(x, edge_index, batch, W1, b1, W2, b2, fc1_W, fc1_b, fc2_W, fc2_b):
    src = edge_index[0].astype(jnp.int32)
    dst = edge_index[1].astype(jnp.int32)
    pad_e = jnp.full((EP - E,), DUMMY, jnp.int32)
    srcp = jnp.concatenate([src, pad_e]).reshape(EPM, 128)
    dstp = jnp.concatenate([dst, pad_e]).reshape(EPM, 128)
    x_pad = jnp.pad(x, ((0, NP - N), (0, 0)))
    batch3 = jnp.concatenate(
        [batch.astype(jnp.int32), jnp.full((NP - N,), G, jnp.int32)]
    ).reshape(NP // 512, 1, 512)
    zeros1 = jnp.zeros((NP,), F32)
    zeros16 = jnp.zeros((NP, 16), F32)
    ones_rows = jnp.ones((K, 128), F32)

    dega, degb = _sc_degree(dstp, zeros1, ones_rows)
    dis, xws1 = _tc_prolog(dega, degb, x_pad, W1)
    acc1a, acc1b = _sc_edge_pass(xws1, srcp, dstp, zeros16, 16)
    h1s = _tc_layer1(acc1a, acc1b, xws1, dis, b1)
    acc2a, acc2b = _sc_edge_pass(h1s, srcp, dstp, zeros16, 16)
    out = _tc_final(acc2a, acc2b, h1s, dis, W2, b2, batch3,
                    fc1_W, fc1_b, fc2_W, fc2_b)
    return out.reshape((G,))


# speedup vs baseline: 1.0581x; 1.0581x over previous
import functools

import jax
import jax.numpy as jnp
from jax import lax
from jax.experimental import pallas as pl
from jax.experimental.pallas import tpu as pltpu
from jax.experimental.pallas import tpu_sc as plsc

N = 100000
NP = 102400
E = 3200000
EP = 3276800
EPM = EP // 128
G = 64
NC, NS = 2, 16
NTILES = NC * NS
K = 8
NSL = NP // NS
DUMMY = NP - 1
EDGE_SPLIT = (1208, 392)

F32 = jnp.float32


def _sc_mesh():
    return plsc.VectorSubcoreMesh(core_axis_name="c", subcore_axis_name="s")


_SC_PARAMS = pltpu.CompilerParams(use_tc_tiling_on_sc=False)


def _sc_degree(dstp, zeros1, ones_rows):
    @functools.partial(
        pl.kernel,
        out_type=[jax.ShapeDtypeStruct((NP,), F32) for _ in range(NC)],
        mesh=_sc_mesh(),
        compiler_params=_SC_PARAMS,
        scratch_types=[
            pltpu.VMEM_SHARED((NP,), F32),
            pltpu.VMEM((K, 128), jnp.int32),
            pltpu.VMEM((K, 128), F32),
            pltpu.SemaphoreType.DMA,
        ],
    )
    def k(dstp_h, z_h, ones_h, dega_h, degb_h, acc, idxd, ones_v, ssem):
        c = lax.axis_index("c")
        s = lax.axis_index("s")
        sl = pl.ds(s * NSL, NSL)
        pltpu.sync_copy(z_h.at[sl], acc.at[sl])
        pltpu.sync_copy(ones_h, ones_v)
        plsc.subcore_barrier()
        tb = (c * NS + s) * (EPM // NTILES)

        def body(kk, carry):
            base = tb + kk * K
            pltpu.sync_copy(dstp_h.at[pl.ds(base, K)], idxd)
            cps = [
                pltpu.async_copy(ones_v.at[j], acc.at[idxd.at[j]], ssem, add=True)
                for j in range(K)
            ]
            for cp in cps:
                cp.wait()
            return carry

        lax.fori_loop(0, EPM // NTILES // K, body, 0)
        plsc.subcore_barrier()

        @pl.when(c == 0)
        def _():
            pltpu.sync_copy(acc.at[sl], dega_h.at[sl])

        @pl.when(c == 1)
        def _():
            pltpu.sync_copy(acc.at[sl], degb_h.at[sl])

    return k(dstp, zeros1, ones_rows)


def _sc_edge_pass(table, srcp, dstp, zeros, R):
    KC = 4
    R0, R1 = EDGE_SPLIT
    assert 16 * (R0 + R1) == EPM and R0 % 8 == 0 and R1 % 8 == 0

    @functools.partial(
        pl.kernel,
        out_type=[jax.ShapeDtypeStruct((NP, R), F32) for _ in range(NC)],
        mesh=_sc_mesh(),
        compiler_params=_SC_PARAMS,
        scratch_types=[
            pltpu.VMEM_SHARED((NP, R), F32),
            [pltpu.VMEM((KC, 128), jnp.int32) for _ in range(2)],
            [pltpu.VMEM((KC, 128), jnp.int32) for _ in range(2)],
            [pltpu.VMEM((KC, 128, R), F32) for _ in range(2)],
            pltpu.SemaphoreType.DMA,
            pltpu.SemaphoreType.DMA,
            pltpu.SemaphoreType.DMA,
        ],
    )
    def k(table_h, srcp_h, dstp_h, z_h, outa_h, outb_h,
          acc, idxs, idxd, rows, gsem, ssem, isem):
        c = lax.axis_index("c")
        s = lax.axis_index("s")
        sl = pl.ds(s * NSL, NSL)
        pltpu.sync_copy(z_h.at[sl], acc.at[sl])
        plsc.subcore_barrier()
        tb = jnp.where(c == 0, s * R0, NS * R0 + s * R1)
        nch = jnp.where(c == 0, R0 // KC, R1 // KC)

        def fire_idx(kk, b):
            base = tb + kk * KC
            pltpu.async_copy(srcp_h.at[pl.ds(base, KC)], idxs[b], isem)
            pltpu.async_copy(dstp_h.at[pl.ds(base, KC)], idxd[b], isem)

        def wait_idx(b):
            pltpu.make_async_copy(srcp_h.at[pl.ds(0, KC)], idxs[b], isem).wait()
            pltpu.make_async_copy(dstp_h.at[pl.ds(0, KC)], idxd[b], isem).wait()

        def fire_gather(b):
            for j in range(KC):
                pltpu.async_copy(table_h.at[idxs[b].at[j]], rows[b].at[j], gsem)

        def wait_gather(b):
            for j in range(KC):
                pltpu.make_async_copy(
                    table_h.at[idxs[b].at[j]], rows[b].at[j], gsem).wait()

        def fire_scatter(b):
            for j in range(KC):
                pltpu.async_copy(rows[b].at[j], acc.at[idxd[b].at[j]], ssem,
                                 add=True)

        def wait_scatter(b):
            for j in range(KC):
                pltpu.make_async_copy(
                    rows[b].at[j], acc.at[idxd[b].at[j]], ssem).wait()

        pltpu.sync_copy(srcp_h.at[pl.ds(tb, KC)], idxs[0])
        pltpu.sync_copy(dstp_h.at[pl.ds(tb, KC)], idxd[0])
        fire_gather(0)

        def phase(kk, b):
            @pl.when(kk >= 1)
            def _():
                wait_scatter(b ^ 1)

            @pl.when(kk + 1 < nch)
            def _():
                fire_idx(kk + 1, b ^ 1)

            wait_gather(b)
            fire_scatter(b)

            @pl.when(kk + 1 < nch)
            def _():
                wait_idx(b ^ 1)
                fire_gather(b ^ 1)

        def body(i, carry):
            phase(2 * i, 0)
            phase(2 * i + 1, 1)
            return carry

        lax.fori_loop(0, nch // 2, body, 0)
        wait_scatter(1)
        plsc.subcore_barrier()

        @pl.when(c == 0)
        def _():
            pltpu.sync_copy(acc.at[sl], outa_h.at[sl])

        @pl.when(c == 1)
        def _():
            pltpu.sync_copy(acc.at[sl], outb_h.at[sl])

    return k(table, srcp, dstp, zeros)


def _tc_prolog(dega, degb, x_pad, W1):
    BB = 2048

    def body(da, db, xr, w1, dis_r, xws_r):
        deg = da[...] + db[...] + 1.0
        dis = lax.rsqrt(deg)
        dis_r[...] = dis
        xw = xr[:, 0:1] * w1[0:1, :] + xr[:, 1:2] * w1[1:2, :]
        xws_r[...] = xw * dis

    return pl.pallas_call(
        body,
        grid=(NP // BB,),
        in_specs=[
            pl.BlockSpec((BB, 1), lambda i: (i, 0)),
            pl.BlockSpec((BB, 1), lambda i: (i, 0)),
            pl.BlockSpec((BB, 2), lambda i: (i, 0)),
            pl.BlockSpec((2, 16), lambda i: (0, 0)),
        ],
        out_specs=[
            pl.BlockSpec((BB, 1), lambda i: (i, 0)),
            pl.BlockSpec((BB, 16), lambda i: (i, 0)),
        ],
        out_shape=[
            jax.ShapeDtypeStruct((NP, 1), F32),
            jax.ShapeDtypeStruct((NP, 16), F32),
        ],
    )(dega.reshape(NP, 1), degb.reshape(NP, 1), x_pad, W1)


def _tc_layer1(acc1a, acc1b, xws1, dis, b1):
    BB = 2048

    def body(aa, ab, xws_r, dis_r, b1r, out_r):
        t = aa[...] + ab[...] + xws_r[...]
        h1 = jnp.maximum(t * dis_r[...] + b1r[...], 0.0)
        out_r[...] = h1 * dis_r[...]

    return pl.pallas_call(
        body,
        grid=(NP // BB,),
        in_specs=[
            pl.BlockSpec((BB, 16), lambda i: (i, 0)),
            pl.BlockSpec((BB, 16), lambda i: (i, 0)),
            pl.BlockSpec((BB, 16), lambda i: (i, 0)),
            pl.BlockSpec((BB, 1), lambda i: (i, 0)),
            pl.BlockSpec((1, 16), lambda i: (0, 0)),
        ],
        out_specs=pl.BlockSpec((BB, 16), lambda i: (i, 0)),
        out_shape=jax.ShapeDtypeStruct((NP, 16), F32),
    )(acc1a, acc1b, xws1, dis, b1.reshape(1, 16))


def _tc_final(acc2a, acc2b, h1s, dis, W2, b2, batch3, fc1_W, fc1_b, fc2_W, fc2_b):
    BF = 512
    NBF = NP // BF

    def body(aa, ab, h1s_r, dis_r, w2, b2r, bat, f1w, f1b, f2w, f2b,
             out_r, sums, counts):
        i = pl.program_id(0)

        @pl.when(i == 0)
        def _():
            sums[...] = jnp.zeros_like(sums)
            counts[...] = jnp.zeros_like(counts)

        m = aa[...] + ab[...] + h1s_r[...]
        h2 = jnp.maximum(
            jnp.dot(m, w2[...], preferred_element_type=F32) * dis_r[...]
            + b2r[...], 0.0)
        bid = bat[0, 0, :]
        oh = (bid[None, :] == lax.broadcasted_iota(jnp.int32, (G, BF), 0)
              ).astype(F32)
        sums[...] += jnp.dot(oh, h2, preferred_element_type=F32)
        counts[...] += jnp.sum(oh, axis=1, keepdims=True)

        @pl.when(i == NBF - 1)
        def _():
            pooled = sums[...] / jnp.maximum(counts[...], 1.0)
            hh = jnp.maximum(
                jnp.dot(pooled, f1w[...], preferred_element_type=F32)
                + f1b[...], 0.0)
            out_r[...] = jnp.dot(hh, f2w[...], preferred_element_type=F32) + f2b[...]

    return pl.pallas_call(
        body,
        grid=(NBF,),
        in_specs=[
            pl.BlockSpec((BF, 16), lambda i: (i, 0)),
            pl.BlockSpec((BF, 16), lambda i: (i, 0)),
            pl.BlockSpec((BF, 16), lambda i: (i, 0)),
            pl.BlockSpec((BF, 1), lambda i: (i, 0)),
            pl.BlockSpec((16, 32), lambda i: (0, 0)),
            pl.BlockSpec((1, 32), lambda i: (0, 0)),
            pl.BlockSpec((1, 1, BF), lambda i: (i, 0, 0)),
            pl.BlockSpec((32, 16), lambda i: (0, 0)),
            pl.BlockSpec((1, 16), lambda i: (0, 0)),
            pl.BlockSpec((16, 1), lambda i: (0, 0)),
            pl.BlockSpec((1, 1), lambda i: (0, 0)),
        ],
        out_specs=pl.BlockSpec((G, 1), lambda i: (0, 0)),
        out_shape=jax.ShapeDtypeStruct((G, 1), F32),
        scratch_shapes=[
            pltpu.VMEM((G, 32), F32),
            pltpu.VMEM((G, 1), F32),
        ],
    )(acc2a, acc2b, h1s, dis, W2, b2.reshape(1, 32), batch3,
      fc1_W, fc1_b.reshape(1, 16), fc2_W, fc2_b.reshape(1, 1))


def kernel(x, edge_index, batch, W1, b1, W2, b2, fc1_W, fc1_b, fc2_W, fc2_b):
    src = edge_index[0].astype(jnp.int32)
    dst = edge_index[1].astype(jnp.int32)
    pad_e = jnp.full((EP - E,), DUMMY, jnp.int32)
    srcp = jnp.concatenate([src, pad_e]).reshape(EPM, 128)
    dstp = jnp.concatenate([dst, pad_e]).reshape(EPM, 128)
    x_pad = jnp.pad(x, ((0, NP - N), (0, 0)))
    batch3 = jnp.concatenate(
        [batch.astype(jnp.int32), jnp.full((NP - N,), G, jnp.int32)]
    ).reshape(NP // 512, 1, 512)
    zeros1 = jnp.zeros((NP,), F32)
    zeros16 = jnp.zeros((NP, 16), F32)
    ones_rows = jnp.ones((K, 128), F32)

    dega, degb = _sc_degree(dstp, zeros1, ones_rows)
    dis, xws1 = _tc_prolog(dega, degb, x_pad, W1)
    acc1a, acc1b = _sc_edge_pass(xws1, srcp, dstp, zeros16, 16)
    h1s = _tc_layer1(acc1a, acc1b, xws1, dis, b1)
    acc2a, acc2b = _sc_edge_pass(h1s, srcp, dstp, zeros16, 16)
    out = _tc_final(acc2a, acc2b, h1s, dis, W2, b2, batch3,
                    fc1_W, fc1_b, fc2_W, fc2_b)
    return out.reshape((G,))


# speedup vs baseline: 1.0937x; 1.0336x over previous
import functools

import jax
import jax.numpy as jnp
from jax import lax
from jax.experimental import pallas as pl
from jax.experimental.pallas import tpu as pltpu
from jax.experimental.pallas import tpu_sc as plsc

N = 100000
NP = 102400
E = 3200000
EP = 3276800
EPM = EP // 128
G = 64
NC, NS = 2, 16
NTILES = NC * NS
K = 8
NSL = NP // NS
DUMMY = NP - 1
EDGE_SPLIT = (1320, 280)
DEG_SPLIT = (920, 680)

F32 = jnp.float32


def _sc_mesh():
    return plsc.VectorSubcoreMesh(core_axis_name="c", subcore_axis_name="s")


_SC_PARAMS = pltpu.CompilerParams(use_tc_tiling_on_sc=False)


def _sc_degree(dstp, zeros1, ones_rows):
    @functools.partial(
        pl.kernel,
        out_type=[jax.ShapeDtypeStruct((NP,), F32) for _ in range(NC)],
        mesh=_sc_mesh(),
        compiler_params=_SC_PARAMS,
        scratch_types=[
            pltpu.VMEM_SHARED((NP,), F32),
            pltpu.VMEM((K, 128), jnp.int32),
            pltpu.VMEM((K, 128), F32),
            pltpu.SemaphoreType.DMA,
        ],
    )
    def k(dstp_h, z_h, ones_h, dega_h, degb_h, acc, idxd, ones_v, ssem):
        c = lax.axis_index("c")
        s = lax.axis_index("s")
        sl = pl.ds(s * NSL, NSL)
        pltpu.sync_copy(z_h.at[sl], acc.at[sl])
        pltpu.sync_copy(ones_h, ones_v)
        plsc.subcore_barrier()
        D0, D1 = DEG_SPLIT
        tb = jnp.where(c == 0, s * D0, NS * D0 + s * D1)
        nch = jnp.where(c == 0, D0 // K, D1 // K)

        def body(kk, carry):
            base = tb + kk * K
            pltpu.sync_copy(dstp_h.at[pl.ds(base, K)], idxd)
            cps = [
                pltpu.async_copy(ones_v.at[j], acc.at[idxd.at[j]], ssem, add=True)
                for j in range(K)
            ]
            for cp in cps:
                cp.wait()
            return carry

        lax.fori_loop(0, nch, body, 0)
        plsc.subcore_barrier()

        @pl.when(c == 0)
        def _():
            pltpu.sync_copy(acc.at[sl], dega_h.at[sl])

        @pl.when(c == 1)
        def _():
            pltpu.sync_copy(acc.at[sl], degb_h.at[sl])

    return k(dstp, zeros1, ones_rows)


def _sc_edge_pass(table, srcp, dstp, zeros, R):
    KC = 4
    R0, R1 = EDGE_SPLIT
    assert 16 * (R0 + R1) == EPM and R0 % 8 == 0 and R1 % 8 == 0

    @functools.partial(
        pl.kernel,
        out_type=[jax.ShapeDtypeStruct((NP, R), F32) for _ in range(NC)],
        mesh=_sc_mesh(),
        compiler_params=_SC_PARAMS,
        scratch_types=[
            pltpu.VMEM_SHARED((NP, R), F32),
            [pltpu.VMEM((KC, 128), jnp.int32) for _ in range(2)],
            [pltpu.VMEM((KC, 128), jnp.int32) for _ in range(2)],
            [pltpu.VMEM((KC, 128, R), F32) for _ in range(2)],
            pltpu.SemaphoreType.DMA,
            pltpu.SemaphoreType.DMA,
            pltpu.SemaphoreType.DMA,
        ],
    )
    def k(table_h, srcp_h, dstp_h, z_h, outa_h, outb_h,
          acc, idxs, idxd, rows, gsem, ssem, isem):
        c = lax.axis_index("c")
        s = lax.axis_index("s")
        sl = pl.ds(s * NSL, NSL)
        pltpu.sync_copy(z_h.at[sl], acc.at[sl])
        plsc.subcore_barrier()
        tb = jnp.where(c == 0, s * R0, NS * R0 + s * R1)
        nch = jnp.where(c == 0, R0 // KC, R1 // KC)

        def fire_idx(kk, b):
            base = tb + kk * KC
            pltpu.async_copy(srcp_h.at[pl.ds(base, KC)], idxs[b], isem)
            pltpu.async_copy(dstp_h.at[pl.ds(base, KC)], idxd[b], isem)

        def wait_idx(b):
            pltpu.make_async_copy(srcp_h.at[pl.ds(0, KC)], idxs[b], isem).wait()
            pltpu.make_async_copy(dstp_h.at[pl.ds(0, KC)], idxd[b], isem).wait()

        def fire_gather(b):
            for j in range(KC):
                pltpu.async_copy(table_h.at[idxs[b].at[j]], rows[b].at[j], gsem)

        def wait_gather(b):
            for j in range(KC):
                pltpu.make_async_copy(
                    table_h.at[idxs[b].at[j]], rows[b].at[j], gsem).wait()

        def fire_scatter(b):
            for j in range(KC):
                pltpu.async_copy(rows[b].at[j], acc.at[idxd[b].at[j]], ssem,
                                 add=True)

        def wait_scatter(b):
            for j in range(KC):
                pltpu.make_async_copy(
                    rows[b].at[j], acc.at[idxd[b].at[j]], ssem).wait()

        pltpu.sync_copy(srcp_h.at[pl.ds(tb, KC)], idxs[0])
        pltpu.sync_copy(dstp_h.at[pl.ds(tb, KC)], idxd[0])
        fire_gather(0)

        def phase(kk, b):
            @pl.when(kk >= 1)
            def _():
                wait_scatter(b ^ 1)

            @pl.when(kk + 1 < nch)
            def _():
                fire_idx(kk + 1, b ^ 1)

            wait_gather(b)
            fire_scatter(b)

            @pl.when(kk + 1 < nch)
            def _():
                wait_idx(b ^ 1)
                fire_gather(b ^ 1)

        def body(i, carry):
            phase(2 * i, 0)
            phase(2 * i + 1, 1)
            return carry

        lax.fori_loop(0, nch // 2, body, 0)
        wait_scatter(1)
        plsc.subcore_barrier()

        @pl.when(c == 0)
        def _():
            pltpu.sync_copy(acc.at[sl], outa_h.at[sl])

        @pl.when(c == 1)
        def _():
            pltpu.sync_copy(acc.at[sl], outb_h.at[sl])

    return k(table, srcp, dstp, zeros)


def _tc_prolog(dega, degb, x_pad, W1):
    BB = 2048

    def body(da, db, xr, w1, dis_r, xws_r):
        deg = da[...] + db[...] + 1.0
        dis = lax.rsqrt(deg)
        dis_r[...] = dis
        xw = xr[:, 0:1] * w1[0:1, :] + xr[:, 1:2] * w1[1:2, :]
        xws_r[...] = xw * dis

    return pl.pallas_call(
        body,
        grid=(NP // BB,),
        in_specs=[
            pl.BlockSpec((BB, 1), lambda i: (i, 0)),
            pl.BlockSpec((BB, 1), lambda i: (i, 0)),
            pl.BlockSpec((BB, 2), lambda i: (i, 0)),
            pl.BlockSpec((2, 16), lambda i: (0, 0)),
        ],
        out_specs=[
            pl.BlockSpec((BB, 1), lambda i: (i, 0)),
            pl.BlockSpec((BB, 16), lambda i: (i, 0)),
        ],
        out_shape=[
            jax.ShapeDtypeStruct((NP, 1), F32),
            jax.ShapeDtypeStruct((NP, 16), F32),
        ],
    )(dega.reshape(NP, 1), degb.reshape(NP, 1), x_pad, W1)


def _tc_layer1(acc1a, acc1b, xws1, dis, b1):
    BB = 2048

    def body(aa, ab, xws_r, dis_r, b1r, out_r):
        t = aa[...] + ab[...] + xws_r[...]
        h1 = jnp.maximum(t * dis_r[...] + b1r[...], 0.0)
        out_r[...] = h1 * dis_r[...]

    return pl.pallas_call(
        body,
        grid=(NP // BB,),
        in_specs=[
            pl.BlockSpec((BB, 16), lambda i: (i, 0)),
            pl.BlockSpec((BB, 16), lambda i: (i, 0)),
            pl.BlockSpec((BB, 16), lambda i: (i, 0)),
            pl.BlockSpec((BB, 1), lambda i: (i, 0)),
            pl.BlockSpec((1, 16), lambda i: (0, 0)),
        ],
        out_specs=pl.BlockSpec((BB, 16), lambda i: (i, 0)),
        out_shape=jax.ShapeDtypeStruct((NP, 16), F32),
    )(acc1a, acc1b, xws1, dis, b1.reshape(1, 16))


def _tc_final(acc2a, acc2b, h1s, dis, W2, b2, batch3, fc1_W, fc1_b, fc2_W, fc2_b):
    BF = 512
    NBF = NP // BF

    def body(aa, ab, h1s_r, dis_r, w2, b2r, bat, f1w, f1b, f2w, f2b,
             out_r, sums, counts):
        i = pl.program_id(0)

        @pl.when(i == 0)
        def _():
            sums[...] = jnp.zeros_like(sums)
            counts[...] = jnp.zeros_like(counts)

        m = aa[...] + ab[...] + h1s_r[...]
        h2 = jnp.maximum(
            jnp.dot(m, w2[...], preferred_element_type=F32) * dis_r[...]
            + b2r[...], 0.0)
        bid = bat[0, 0, :]
        oh = (bid[None, :] == lax.broadcasted_iota(jnp.int32, (G, BF), 0)
              ).astype(F32)
        sums[...] += jnp.dot(oh, h2, preferred_element_type=F32)
        counts[...] += jnp.sum(oh, axis=1, keepdims=True)

        @pl.when(i == NBF - 1)
        def _():
            pooled = sums[...] / jnp.maximum(counts[...], 1.0)
            hh = jnp.maximum(
                jnp.dot(pooled, f1w[...], preferred_element_type=F32)
                + f1b[...], 0.0)
            out_r[...] = jnp.dot(hh, f2w[...], preferred_element_type=F32) + f2b[...]

    return pl.pallas_call(
        body,
        grid=(NBF,),
        in_specs=[
            pl.BlockSpec((BF, 16), lambda i: (i, 0)),
            pl.BlockSpec((BF, 16), lambda i: (i, 0)),
            pl.BlockSpec((BF, 16), lambda i: (i, 0)),
            pl.BlockSpec((BF, 1), lambda i: (i, 0)),
            pl.BlockSpec((16, 32), lambda i: (0, 0)),
            pl.BlockSpec((1, 32), lambda i: (0, 0)),
            pl.BlockSpec((1, 1, BF), lambda i: (i, 0, 0)),
            pl.BlockSpec((32, 16), lambda i: (0, 0)),
            pl.BlockSpec((1, 16), lambda i: (0, 0)),
            pl.BlockSpec((16, 1), lambda i: (0, 0)),
            pl.BlockSpec((1, 1), lambda i: (0, 0)),
        ],
        out_specs=pl.BlockSpec((G, 1), lambda i: (0, 0)),
        out_shape=jax.ShapeDtypeStruct((G, 1), F32),
        scratch_shapes=[
            pltpu.VMEM((G, 32), F32),
            pltpu.VMEM((G, 1), F32),
        ],
    )(acc2a, acc2b, h1s, dis, W2, b2.reshape(1, 32), batch3,
      fc1_W, fc1_b.reshape(1, 16), fc2_W, fc2_b.reshape(1, 1))


def kernel(x, edge_index, batch, W1, b1, W2, b2, fc1_W, fc1_b, fc2_W, fc2_b):
    src = edge_index[0].astype(jnp.int32)
    dst = edge_index[1].astype(jnp.int32)
    pad_e = jnp.full((EP - E,), DUMMY, jnp.int32)
    srcp = jnp.concatenate([src, pad_e]).reshape(EPM, 128)
    dstp = jnp.concatenate([dst, pad_e]).reshape(EPM, 128)
    x_pad = jnp.pad(x, ((0, NP - N), (0, 0)))
    batch3 = jnp.concatenate(
        [batch.astype(jnp.int32), jnp.full((NP - N,), G, jnp.int32)]
    ).reshape(NP // 512, 1, 512)
    zeros1 = jnp.zeros((NP,), F32)
    zeros16 = jnp.zeros((NP, 16), F32)
    ones_rows = jnp.ones((K, 128), F32)

    dega, degb = _sc_degree(dstp, zeros1, ones_rows)
    dis, xws1 = _tc_prolog(dega, degb, x_pad, W1)
    acc1a, acc1b = _sc_edge_pass(xws1, srcp, dstp, zeros16, 16)
    h1s = _tc_layer1(acc1a, acc1b, xws1, dis, b1)
    acc2a, acc2b = _sc_edge_pass(h1s, srcp, dstp, zeros16, 16)
    out = _tc_final(acc2a, acc2b, h1s, dis, W2, b2, batch3,
                    fc1_W, fc1_b, fc2_W, fc2_b)
    return out.reshape((G,))
